# Initial kernel scaffold; baseline (speedup 1.0000x reference)
import jax
import jax.numpy as jnp
from jax.experimental import pallas as pl


def kernel(tensor, weight):
    raise NotImplementedError("write your pallas kernel here")


# speedup vs baseline: 1.4595x; 1.4595x over previous
import functools

import jax
import jax.numpy as jnp
from jax import lax
from jax.experimental import pallas as pl
from jax.experimental.pallas import tpu as pltpu
from jax.experimental.pallas import tpu_sc as plsc

D = 32
NC, NS = 2, 16
NW = NC * NS
SEG = 128
K = 8
CHUNK = SEG * K


@functools.partial(jax.jit, static_argnames=("total",))
def _gather_rows(idx2d, table, total):
    steps = total // (NW * CHUNK)
    per_w = total // NW
    mesh = plsc.VectorSubcoreMesh(
        core_axis_name="c", subcore_axis_name="s", num_cores=NC, num_subcores=NS
    )

    @functools.partial(
        pl.kernel,
        out_type=jax.ShapeDtypeStruct((total, D), jnp.float32),
        mesh=mesh,
        scratch_types=[
            pltpu.VMEM((K, SEG), jnp.int32),
            pltpu.VMEM((CHUNK, D), jnp.float32),
            pltpu.SemaphoreType.DMA,
        ],
        compiler_params=pltpu.CompilerParams(use_tc_tiling_on_sc=False),
    )
    def emb(idx_hbm, table_hbm, out_hbm, idx_v, rows_v, sem):
        wid = lax.axis_index("s") * NC + lax.axis_index("c")
        base = wid * per_w

        def body(step, carry):
            off = pl.multiple_of(base + step * CHUNK, CHUNK)
            pltpu.sync_copy(idx_hbm.at[pl.ds(pl.multiple_of(off // SEG, K), K), :], idx_v)
            copies = []
            for j in range(K):
                copies.append(
                    pltpu.async_copy(
                        table_hbm.at[idx_v.at[j]],
                        rows_v.at[pl.ds(j * SEG, SEG), :],
                        sem,
                    )
                )
            for c in copies:
                c.wait()
            pltpu.sync_copy(rows_v, out_hbm.at[pl.ds(off, CHUNK), :])
            return carry

        lax.fori_loop(0, steps, body, 0)

    return emb(idx2d, table)


def kernel(tensor, weight):
    shape = tensor.shape
    total = tensor.size
    idx2d = tensor.astype(jnp.int32).reshape(total // SEG, SEG)
    out = _gather_rows(idx2d, weight, total)
    return out.reshape(*shape, D)


# speedup vs baseline: 1.5021x; 1.0291x over previous
import functools

import jax
import jax.numpy as jnp
from jax import lax
from jax.experimental import pallas as pl
from jax.experimental.pallas import tpu as pltpu
from jax.experimental.pallas import tpu_sc as plsc

D = 32
NC, NS = 2, 16
NW = NC * NS
SEG = 128
K = 8
CHUNK = SEG * K
NBUF = 3


@functools.partial(jax.jit, static_argnames=("total",))
def _gather_rows(idx2d, table, total):
    per_w = total // NW
    steps = per_w // CHUNK
    segs_per_w = per_w // SEG
    outer = (steps + NBUF - 1) // NBUF
    mesh = plsc.VectorSubcoreMesh(
        core_axis_name="c", subcore_axis_name="s", num_cores=NC, num_subcores=NS
    )

    @functools.partial(
        pl.kernel,
        out_type=jax.ShapeDtypeStruct((total, D), jnp.float32),
        mesh=mesh,
        scratch_types=[
            pltpu.VMEM((segs_per_w, SEG), jnp.int32),
            pltpu.VMEM((NBUF, CHUNK, D), jnp.float32),
            [pltpu.SemaphoreType.DMA] * NBUF,
            [pltpu.SemaphoreType.DMA] * NBUF,
        ],
        compiler_params=pltpu.CompilerParams(use_tc_tiling_on_sc=False),
    )
    def emb(idx_hbm, table_hbm, out_hbm, idx_v, rows_v, gsems, ssems):
        wid = lax.axis_index("s") * NC + lax.axis_index("c")
        base = wid * per_w

        row0 = pl.multiple_of(wid * segs_per_w, 8)
        pltpu.sync_copy(idx_hbm.at[pl.ds(row0, segs_per_w), :], idx_v)

        def fire(s, b):
            for j in range(K):
                pltpu.async_copy(
                    table_hbm.at[idx_v.at[s * K + j]],
                    rows_v.at[b, pl.ds(j * SEG, SEG), :],
                    gsems[b],
                )

        def drain_gather(b):
            pltpu.make_async_copy(
                out_hbm.at[pl.ds(0, CHUNK), :], rows_v.at[0], gsems[b]
            ).wait()

        def drain_store(b):
            pltpu.make_async_copy(
                out_hbm.at[pl.ds(0, CHUNK), :], rows_v.at[0], ssems[b]
            ).wait()

        for b in range(NBUF):
            fire(b, b)

        def body(g, carry):
            for b in range(NBUF):
                s = g * NBUF + b

                @pl.when(s < steps)
                def _():
                    drain_gather(b)
                    off = pl.multiple_of(base + s * CHUNK, CHUNK)
                    pltpu.async_copy(
                        rows_v.at[b], out_hbm.at[pl.ds(off, CHUNK), :], ssems[b]
                    )

                    @pl.when(s + NBUF < steps)
                    def _():
                        drain_store(b)
                        fire(s + NBUF, b)

            return carry

        lax.fori_loop(0, outer, body, 0)
        for b in range(NBUF):
            drain_store(b)

    return emb(idx2d, table)


def kernel(tensor, weight):
    shape = tensor.shape
    total = tensor.size
    idx2d = tensor.astype(jnp.int32).reshape(total // SEG, SEG)
    out = _gather_rows(idx2d, weight, total)
    return out.reshape(*shape, D)
